# Initial kernel scaffold; baseline (speedup 1.0000x reference)
import jax
import jax.numpy as jnp
from jax.experimental import pallas as pl


def kernel(x, edge_index, batch, W1, W2, Wfc):
    raise NotImplementedError("write your pallas kernel here")


# speedup vs baseline: 10.2292x; 10.2292x over previous
import functools

import jax
import jax.numpy as jnp
from jax import lax
from jax.experimental import pallas as pl
from jax.experimental.pallas import tpu as pltpu
from jax.experimental.pallas import tpu_sc as plsc

N = 10000
E = 320000
F_IN = 128
DIM = 64
G = 64

NC = 2
NS = 16
NW = NC * NS
CH = 128
RPW = 79
EP = NW * RPW * CH
NP = 10240
NPT = NP // NS
NDUMMY = N

_mesh = plsc.VectorSubcoreMesh(
    core_axis_name="c", subcore_axis_name="s", num_cores=NC, num_subcores=NS
)


def _edge64_body(b_tab, src3, dst3, z64, z1, agg_out, cnt_out,
                 src_v, dst_v, rows_v, ones_v, agg_sp, cnt_sp, sem):
  c = lax.axis_index("c")
  s = lax.axis_index("s")
  w = s * NC + c
  pltpu.sync_copy(z64, agg_sp.at[pl.ds(s * NPT, NPT), :])
  pltpu.sync_copy(z1, cnt_sp.at[pl.ds(s * NPT, NPT)])
  pltpu.sync_copy(src3.at[w], src_v)
  pltpu.sync_copy(dst3.at[w], dst_v)
  for j in range(8):
    ones_v[pl.ds(j * 16, 16)] = jnp.ones((16,), jnp.float32)
  plsc.subcore_barrier()

  def body(j, carry):
    pltpu.async_copy(b_tab.at[src_v.at[j]], rows_v, sem).wait()
    pltpu.sync_copy(ones_v, cnt_sp.at[dst_v.at[j]], add=True)
    pltpu.sync_copy(rows_v, agg_sp.at[dst_v.at[j]], add=True)
    return carry

  lax.fori_loop(0, RPW, body, 0)
  plsc.subcore_barrier()
  pltpu.sync_copy(agg_sp.at[pl.ds(s * NPT, NPT), :],
                  agg_out.at[c, pl.ds(s * NPT, NPT), :])
  pltpu.sync_copy(cnt_sp.at[pl.ds(s * NPT, NPT)],
                  cnt_out.at[c, pl.ds(s * NPT, NPT)])


_sc_params = pltpu.CompilerParams(use_tc_tiling_on_sc=False)

_edge64 = pl.kernel(
    _edge64_body,
    out_type=(
        jax.ShapeDtypeStruct((NC, NP, DIM), jnp.float32),
        jax.ShapeDtypeStruct((NC, NP), jnp.float32),
    ),
    mesh=_mesh,
    compiler_params=_sc_params,
    scratch_types=[
        pltpu.VMEM((RPW, CH), jnp.int32),
        pltpu.VMEM((RPW, CH), jnp.int32),
        pltpu.VMEM((CH, DIM), jnp.float32),
        pltpu.VMEM((CH,), jnp.float32),
        pltpu.VMEM_SHARED((NP, DIM), jnp.float32),
        pltpu.VMEM_SHARED((NP,), jnp.float32),
        pltpu.SemaphoreType.DMA,
    ],
)


def _edge1_body(q_tab, src3, dst3, z1, agg_out,
                src_v, dst_v, vals_v, agg_sp, sem):
  c = lax.axis_index("c")
  s = lax.axis_index("s")
  w = s * NC + c
  pltpu.sync_copy(z1, agg_sp.at[pl.ds(s * NPT, NPT)])
  pltpu.sync_copy(src3.at[w], src_v)
  pltpu.sync_copy(dst3.at[w], dst_v)
  plsc.subcore_barrier()

  def body(j, carry):
    pltpu.async_copy(q_tab.at[src_v.at[j]], vals_v, sem).wait()
    pltpu.sync_copy(vals_v, agg_sp.at[dst_v.at[j]], add=True)
    return carry

  lax.fori_loop(0, RPW, body, 0)
  plsc.subcore_barrier()
  pltpu.sync_copy(agg_sp.at[pl.ds(s * NPT, NPT)],
                  agg_out.at[c, pl.ds(s * NPT, NPT)])


_edge1 = pl.kernel(
    _edge1_body,
    out_type=jax.ShapeDtypeStruct((NC, NP), jnp.float32),
    mesh=_mesh,
    compiler_params=_sc_params,
    scratch_types=[
        pltpu.VMEM((RPW, CH), jnp.int32),
        pltpu.VMEM((RPW, CH), jnp.int32),
        pltpu.VMEM((CH,), jnp.float32),
        pltpu.VMEM_SHARED((NP,), jnp.float32),
        pltpu.SemaphoreType.DMA,
    ],
)


def _mm(x, y):
  return lax.dot_general(x, y, (((1,), (0,)), ((), ())),
                         preferred_element_type=jnp.float32)


def _tc_a_body(x_ref, w1_ref, a_ref, b_ref):
  xv = x_ref[...]
  w = w1_ref[...]
  a_ref[...] = _mm(xv, w[0:F_IN, :])
  b_ref[...] = _mm(xv, w[F_IN:2 * F_IN, :])


def _tc_a(xp, w1):
  return pl.pallas_call(
      _tc_a_body,
      out_shape=(
          jax.ShapeDtypeStruct((NP, DIM), jnp.float32),
          jax.ShapeDtypeStruct((NP, DIM), jnp.float32),
      ),
  )(xp, w1)


def _tc_c_body(a_ref, aggp_ref, cntp_ref, w2_ref, wfc_ref,
               p_ref, q_ref, inv_ref):
  cnt = cntp_ref[0] + cntp_ref[1]
  inv = 1.0 / jnp.maximum(cnt, 1.0)
  agg = (aggp_ref[0] + aggp_ref[1]) * inv[:, None]
  h1 = jnp.maximum(a_ref[...] + agg, 0.0)
  wa = _mm(w2_ref[0:DIM, :], wfc_ref[...])[:, 0]
  wb = _mm(w2_ref[DIM:2 * DIM, :], wfc_ref[...])[:, 0]
  p_ref[...] = jnp.sum(h1 * wa[None, :], axis=1)
  q_ref[...] = jnp.sum(h1 * wb[None, :], axis=1)
  inv_ref[...] = inv


def _tc_c(a, aggp, cntp, w2, wfc):
  return pl.pallas_call(
      _tc_c_body,
      out_shape=(
          jax.ShapeDtypeStruct((NP,), jnp.float32),
          jax.ShapeDtypeStruct((NP,), jnp.float32),
          jax.ShapeDtypeStruct((NP,), jnp.float32),
      ),
  )(a, aggp, cntp, w2, wfc)


def _tc_e_body(p_ref, agg2p_ref, inv_ref, batch_ref, out_ref):
  z = p_ref[...] + (agg2p_ref[0] + agg2p_ref[1]) * inv_ref[...]
  b = batch_ref[...]
  oh = (b[:, None] == lax.broadcasted_iota(jnp.int32, (NP, 128), 1))
  oh = oh.astype(jnp.float32)
  pooled = _mm(z[None, :], oh)
  counts = jnp.sum(oh, axis=0)[None, :]
  mean = pooled / jnp.maximum(counts, 1.0)
  out_ref[...] = 1.0 / (1.0 + jnp.exp(-mean))


def _tc_e(p, agg2p, inv, batchp):
  return pl.pallas_call(
      _tc_e_body,
      out_shape=jax.ShapeDtypeStruct((1, 128), jnp.float32),
  )(p, agg2p, inv, batchp)


@jax.jit
def kernel(x, edge_index, batch, W1, W2, Wfc):
  xp = jnp.zeros((NP, F_IN), jnp.float32).at[:N].set(x)
  src = edge_index[0]
  dst = edge_index[1]
  srcp = jnp.zeros((EP,), jnp.int32).at[:E].set(src).reshape(NW, RPW, CH)
  dstp = jnp.full((EP,), NDUMMY, jnp.int32).at[:E].set(dst).reshape(NW, RPW, CH)
  batchp = jnp.full((NP,), G, jnp.int32).at[:N].set(batch)
  z64 = jnp.zeros((NPT, DIM), jnp.float32)
  z1 = jnp.zeros((NPT,), jnp.float32)

  a, b = _tc_a(xp, W1)
  aggp, cntp = _edge64(b, srcp, dstp, z64, z1)
  p, q, inv = _tc_c(a, aggp, cntp, W2, Wfc)
  agg2p = _edge1(q, srcp, dstp, z1)
  out = _tc_e(p, agg2p, inv, batchp)
  return out[0, :G].reshape(G, 1)
